# baseline (speedup 1.0000x reference)
import functools

import jax
import jax.numpy as jnp
from jax import lax
from jax.experimental import pallas as pl
from jax.experimental.pallas import tpu as pltpu
from jax.experimental.pallas import tpu_sc as plsc

D_MODEL = 64
B_SETS = 1024
S_LEN = 200
GAMMA = 1.0
BETA = 0.0

N_TOTAL = B_SETS * S_LEN
NUM_CORES = 2
NUM_SUBCORES = 16
NW = NUM_CORES * NUM_SUBCORES
BW = B_SETS // NW
SCH = 4
NCH = S_LEN // SCH
LANES = 16


@functools.partial(
    pl.kernel,
    mesh=plsc.VectorSubcoreMesh(core_axis_name="c", subcore_axis_name="s"),
    out_type=jax.ShapeDtypeStruct(
        (S_LEN, D_MODEL // 8, B_SETS // 128, 8, 128), jnp.float32),
    scratch_types=[
        pltpu.VMEM((NCH, 4 * BW), jnp.int32),
        pltpu.VMEM((2, SCH * BW, D_MODEL), jnp.float32),
        pltpu.VMEM((2, SCH, D_MODEL // 8, 8, BW + 1), jnp.float32),
        pltpu.SemaphoreType.DMA,
        pltpu.SemaphoreType.DMA,
    ],
    compiler_params=pltpu.CompilerParams(
        use_tc_tiling_on_sc=False, needs_layout_passes=False),
)
def _sc_gather(idx_hbm, table_hbm, out_hbm, idx_v, gbuf, tbuf, gsem0, gsem1):
    wid = lax.axis_index("s") * NUM_CORES + lax.axis_index("c")
    pltpu.sync_copy(idx_hbm.at[wid], idx_v)
    jb = wid // 4
    l0 = (wid % 4) * BW
    sems = (gsem0, gsem1)
    lane = lax.iota(jnp.int32, LANES)
    dl = [[(c * LANES + lane) // 8, (c * LANES + lane) % 8]
          for c in range(D_MODEL // LANES)]

    def issue(k, p):
        pltpu.async_copy(table_hbm.at[idx_v.at[k]], gbuf.at[p], sems[p])

    def drain(p):
        pltpu.make_async_copy(
            table_hbm.at[idx_v.at[0]], gbuf.at[p], sems[p]).wait()

    def store(k, p):
        pltpu.sync_copy(
            tbuf.at[p, :, :, :, pl.ds(0, BW)],
            out_hbm.at[pl.ds(SCH * k, SCH), :, jb, :, pl.ds(l0, BW)])

    def transpose(p):
        g = gbuf.at[p]

        @plsc.parallel_loop(0, SCH * BW, unroll=4)
        def rloop(r):
            s_l = r // BW
            b_l = jnp.full((LANES,), 0, jnp.int32) + (r % BW)
            t = tbuf.at[p, s_l]
            vs = [g[r, pl.ds(c * LANES, LANES)] for c in range(D_MODEL // LANES)]
            for c in range(D_MODEL // LANES):
                plsc.store_scatter(t, [dl[c][0], dl[c][1], b_l], vs[c])

    def step(k, p, kk):
        drain(p)
        transpose(p)

        @pl.when(kk < NCH // 2 - 1)
        def _():
            issue(k + 2, p)

        store(k, p)

    issue(0, 0)
    issue(1, 1)

    def body(kk, carry):
        step(2 * kk, 0, kk)
        step(2 * kk + 1, 1, kk)
        return carry

    lax.fori_loop(0, NCH // 2, body, 0)


_BI = 8
_BL = 512


def _bias_body(pi_ref, pall_ref, out_ref):
    pi = pi_ref[...][:, None, :]
    pall = pall_ref[...][None, :, :]
    neg_abs = (jnp.minimum(pi, pall) - jnp.maximum(pi, pall)).astype(jnp.float32)
    if GAMMA == 1.0 and BETA == 0.0:
        out_ref[...] = neg_abs
    else:
        out_ref[...] = GAMMA * neg_abs + BETA


_bias_t = pl.pallas_call(
    _bias_body,
    grid=(S_LEN // _BI, B_SETS // _BL),
    in_specs=[
        pl.BlockSpec((_BI, _BL), lambda i, b: (i, b)),
        pl.BlockSpec((S_LEN, _BL), lambda i, b: (0, b)),
    ],
    out_specs=pl.BlockSpec((_BI, S_LEN, _BL), lambda i, b: (i, 0, b)),
    out_shape=jax.ShapeDtypeStruct((S_LEN, S_LEN, B_SETS), jnp.float32),
)


def kernel(set_positions, router_emb):
    pos = set_positions.astype(jnp.int32)
    idx3d = (pos.reshape(NW, BW, NCH, SCH)
             .transpose(0, 2, 3, 1)
             .reshape(NW, NCH, SCH * BW))
    lin5 = _sc_gather(idx3d, router_emb)
    desc_router = jnp.transpose(lin5, (2, 4, 0, 1, 3)).reshape(
        B_SETS, S_LEN, D_MODEL)
    pos_t = pos.T
    geom_bias = jnp.transpose(_bias_t(pos_t, pos_t), (2, 0, 1))
    return (desc_router, geom_bias)


# speedup vs baseline: 1.0481x; 1.0481x over previous
import functools

import jax
import jax.numpy as jnp
from jax import lax
from jax.experimental import pallas as pl
from jax.experimental.pallas import tpu as pltpu
from jax.experimental.pallas import tpu_sc as plsc

D_MODEL = 64
B_SETS = 1024
S_LEN = 200
GAMMA = 1.0
BETA = 0.0

N_TOTAL = B_SETS * S_LEN
NUM_CORES = 2
NUM_SUBCORES = 16
NW = NUM_CORES * NUM_SUBCORES
BW = B_SETS // NW
SCH = 4
NCH = S_LEN // SCH
LANES = 16


@functools.partial(
    pl.kernel,
    mesh=plsc.VectorSubcoreMesh(core_axis_name="c", subcore_axis_name="s"),
    out_type=jax.ShapeDtypeStruct(
        (S_LEN, D_MODEL // 8, B_SETS // 128, 8, 128), jnp.float32),
    scratch_types=[
        pltpu.VMEM((NCH, 4 * BW), jnp.int32),
        pltpu.VMEM((2, SCH * BW, D_MODEL), jnp.float32),
        pltpu.VMEM((2, SCH, D_MODEL // 8, 8, BW + 1), jnp.float32),
        pltpu.SemaphoreType.DMA,
        pltpu.SemaphoreType.DMA,
    ],
    compiler_params=pltpu.CompilerParams(
        use_tc_tiling_on_sc=False, needs_layout_passes=False),
)
def _sc_gather(idx_hbm, table_hbm, out_hbm, idx_v, gbuf, tbuf, gsem0, gsem1):
    wid = lax.axis_index("s") * NUM_CORES + lax.axis_index("c")
    pltpu.sync_copy(idx_hbm.at[wid], idx_v)
    jb = wid // 4
    l0 = (wid % 4) * BW
    sems = (gsem0, gsem1)
    lane = lax.iota(jnp.int32, LANES)
    dl = [[(c * LANES + lane) // 8, (c * LANES + lane) % 8]
          for c in range(D_MODEL // LANES)]

    def issue(k, p):
        pltpu.async_copy(table_hbm.at[idx_v.at[k]], gbuf.at[p], sems[p])

    def drain(p):
        pltpu.make_async_copy(
            table_hbm.at[idx_v.at[0]], gbuf.at[p], sems[p]).wait()

    def store(k, p):
        pltpu.sync_copy(
            tbuf.at[p, :, :, :, pl.ds(0, BW)],
            out_hbm.at[pl.ds(SCH * k, SCH), :, jb, :, pl.ds(l0, BW)])

    def transpose(p):
        g = gbuf.at[p]

        @plsc.parallel_loop(0, SCH * BW, unroll=4)
        def rloop(r):
            s_l = r // BW
            b_l = jnp.full((LANES,), 0, jnp.int32) + (r % BW)
            t = tbuf.at[p, s_l]
            vs = [g[r, pl.ds(c * LANES, LANES)] for c in range(D_MODEL // LANES)]
            for c in range(D_MODEL // LANES):
                plsc.store_scatter(t, [dl[c][0], dl[c][1], b_l], vs[c])

    def step(k, p, kk):
        drain(p)
        transpose(p)

        @pl.when(kk < NCH // 2 - 1)
        def _():
            issue(k + 2, p)

        store(k, p)

    issue(0, 0)
    issue(1, 1)

    def body(kk, carry):
        step(2 * kk, 0, kk)
        step(2 * kk + 1, 1, kk)
        return carry

    lax.fori_loop(0, NCH // 2, body, 0)


_BI = 8
_BL = 512


def _bias_body(pi_ref, pall_ref, out_ref):
    pi = pi_ref[...][:, None, :]
    pall = pall_ref[...][None, :, :]
    neg_abs = (jnp.minimum(pi, pall) - jnp.maximum(pi, pall)).astype(jnp.float32)
    if GAMMA == 1.0 and BETA == 0.0:
        out_ref[...] = neg_abs
    else:
        out_ref[...] = GAMMA * neg_abs + BETA


_bias_t = pl.pallas_call(
    _bias_body,
    grid=(S_LEN // _BI, B_SETS // _BL),
    in_specs=[
        pl.BlockSpec((_BI, _BL), lambda i, b: (i, b)),
        pl.BlockSpec((S_LEN, _BL), lambda i, b: (0, b)),
    ],
    out_specs=pl.BlockSpec((_BI, S_LEN, _BL), lambda i, b: (i, 0, b)),
    out_shape=jax.ShapeDtypeStruct((S_LEN, S_LEN, B_SETS), jnp.float32),
)


def kernel(set_positions, router_emb):
    pos = set_positions.astype(jnp.int32)
    idx3d = (pos.reshape(NW, BW, NCH, SCH)
             .transpose(0, 2, 3, 1)
             .reshape(NW, NCH, SCH * BW)) * 2
    table2 = jnp.pad(router_emb, ((0, 0), (0, D_MODEL))).reshape(-1, D_MODEL)
    lin5 = _sc_gather(idx3d, table2)
    desc_router = jnp.transpose(lin5, (2, 4, 0, 1, 3)).reshape(
        B_SETS, S_LEN, D_MODEL)
    pos_t = pos.T
    geom_bias = jnp.transpose(_bias_t(pos_t, pos_t), (2, 0, 1))
    return (desc_router, geom_bias)


# speedup vs baseline: 1.1339x; 1.0818x over previous
import functools

import jax
import jax.numpy as jnp
from jax import lax
from jax.experimental import pallas as pl
from jax.experimental.pallas import tpu as pltpu
from jax.experimental.pallas import tpu_sc as plsc

D_MODEL = 64
B_SETS = 1024
S_LEN = 200
GAMMA = 1.0
BETA = 0.0

N_TOTAL = B_SETS * S_LEN
NUM_CORES = 2
NUM_SUBCORES = 16
NW = NUM_CORES * NUM_SUBCORES
BW = B_SETS // NW
SCH = 4
NCH = S_LEN // SCH
LANES = 16


@functools.partial(
    pl.kernel,
    mesh=plsc.VectorSubcoreMesh(core_axis_name="c", subcore_axis_name="s"),
    out_type=jax.ShapeDtypeStruct(
        (S_LEN, D_MODEL // 8, B_SETS // 128, 8, 128), jnp.float32),
    scratch_types=[
        pltpu.VMEM((NCH, 4 * BW), jnp.int32),
        pltpu.VMEM((2, SCH * BW, D_MODEL), jnp.float32),
        pltpu.VMEM((2, SCH, D_MODEL // 8, 8, BW + 1), jnp.float32),
        pltpu.SemaphoreType.DMA,
        pltpu.SemaphoreType.DMA,
    ],
    compiler_params=pltpu.CompilerParams(
        use_tc_tiling_on_sc=False, needs_layout_passes=False),
)
def _sc_gather(idx_hbm, table_hbm, out_hbm, idx_v, gbuf, tbuf, gsem0, gsem1):
    wid = lax.axis_index("s") * NUM_CORES + lax.axis_index("c")
    pltpu.sync_copy(idx_hbm.at[wid], idx_v)
    jb = wid // 4
    l0 = (wid % 4) * BW
    sems = (gsem0, gsem1)
    lane = lax.iota(jnp.int32, LANES)
    dl = [[(c * LANES + lane) // 8, (c * LANES + lane) % 8]
          for c in range(D_MODEL // LANES)]

    def issue(k, p):
        pltpu.async_copy(table_hbm.at[idx_v.at[k]], gbuf.at[p], sems[p])

    def drain(p):
        pltpu.make_async_copy(
            table_hbm.at[idx_v.at[0]], gbuf.at[p], sems[p]).wait()

    def store(k, p):
        pltpu.sync_copy(
            tbuf.at[p, :, :, :, pl.ds(0, BW)],
            out_hbm.at[pl.ds(SCH * k, SCH), :, jb, :, pl.ds(l0, BW)])

    def transpose(p):
        g = gbuf.at[p]

        @plsc.parallel_loop(0, SCH * BW, unroll=4)
        def rloop(r):
            s_l = r // BW
            b_l = jnp.full((LANES,), 0, jnp.int32) + (r % BW)
            t = tbuf.at[p, s_l]
            vs = [g[r, pl.ds(c * LANES, LANES)] for c in range(D_MODEL // LANES)]
            for c in range(D_MODEL // LANES):
                plsc.store_scatter(t, [dl[c][0], dl[c][1], b_l], vs[c])

    def step(k, p, kk):
        drain(p)
        transpose(p)

        @pl.when(kk < NCH // 2 - 1)
        def _():
            issue(k + 2, p)

        store(k, p)

    issue(0, 0)
    issue(1, 1)

    def body(kk, carry):
        step(2 * kk, 0, kk)
        step(2 * kk + 1, 1, kk)
        return carry

    lax.fori_loop(0, NCH // 2, body, 0)


_BI = 8
_BL = 1024


def _bias_body(pi_ref, pall_ref, out_ref):
    pi = pi_ref[...][:, None, :]
    pall = pall_ref[...][None, :, :]
    neg_abs = (jnp.minimum(pi, pall) - jnp.maximum(pi, pall)).astype(jnp.float32)
    if GAMMA == 1.0 and BETA == 0.0:
        out_ref[...] = neg_abs
    else:
        out_ref[...] = GAMMA * neg_abs + BETA


_bias_t = pl.pallas_call(
    _bias_body,
    grid=(S_LEN // _BI, B_SETS // _BL),
    in_specs=[
        pl.BlockSpec((_BI, _BL), lambda i, b: (i, b)),
        pl.BlockSpec((S_LEN, _BL), lambda i, b: (0, b)),
    ],
    out_specs=pl.BlockSpec((_BI, S_LEN, _BL), lambda i, b: (i, 0, b)),
    out_shape=jax.ShapeDtypeStruct((S_LEN, S_LEN, B_SETS), jnp.float32),
)


def kernel(set_positions, router_emb):
    pos = set_positions.astype(jnp.int32)
    idx3d = (pos.reshape(NW, BW, NCH, SCH)
             .transpose(0, 2, 3, 1)
             .reshape(NW, NCH, SCH * BW)) * 2
    table2 = jnp.pad(router_emb, ((0, 0), (0, D_MODEL))).reshape(-1, D_MODEL)
    lin5 = _sc_gather(idx3d, table2)
    desc_router = jnp.transpose(lin5, (2, 4, 0, 1, 3)).reshape(
        B_SETS, S_LEN, D_MODEL)
    pos_t = pos.T
    geom_bias = jnp.transpose(_bias_t(pos_t, pos_t), (2, 0, 1))
    return (desc_router, geom_bias)
